# baseline (speedup 1.0000x reference)
import jax
import jax.numpy as jnp
from jax import lax
from jax.experimental import pallas as pl
from jax.experimental.pallas import tpu as pltpu
from jax.experimental.pallas import tpu_sc as plsc

H = 128
DEPTH = 4
NC, NS = 2, 16
NW = NC * NS

NB = 320000
NA = 10000
MAX_NB = 32
BPW = 10112
NBP = NW * BPW
APW = 320
NAP = NW * APW

GS_CHUNK = 128
GS_CHUNKS = APW * MAX_NB // GS_CHUNK
GS_RING = 6
ED_CHUNK = 128
ED_CHUNKS = BPW // ED_CHUNK
ED_RING = 3


def _init_mm_kernel(fb_ref, wi_ref, out_ref):
    acc = jnp.dot(fb_ref[...], wi_ref[...], preferred_element_type=jnp.float32)
    out_ref[...] = jnp.maximum(acc, 0.0)


def _reduce_kernel(nei_ref, out_ref):
    x = nei_ref[...]
    br = x.shape[0]
    out_ref[...] = x.reshape(br // MAX_NB, MAX_NB, H).sum(axis=1)


def _layer_mm_kernel(fb_ref, ga_ref, gr_ref, wi_ref, wh_ref, out_ref):
    acc = jnp.dot(fb_ref[...], wi_ref[...], preferred_element_type=jnp.float32)
    wh = wh_ref[...]
    acc = acc + jnp.dot(ga_ref[...].astype(jnp.bfloat16), wh,
                        preferred_element_type=jnp.float32)
    acc = acc - jnp.dot(gr_ref[...].astype(jnp.bfloat16), wh,
                        preferred_element_type=jnp.float32)
    out_ref[...] = jnp.maximum(acc, 0.0)


def _final_mm_kernel(fa_ref, am_ref, wo1_ref, wo2_ref, mask_ref, out_ref):
    acc = jnp.dot(fa_ref[...], wo1_ref[...], preferred_element_type=jnp.float32)
    acc = acc + jnp.dot(am_ref[...].astype(jnp.bfloat16), wo2_ref[...],
                        preferred_element_type=jnp.float32)
    out_ref[...] = jnp.maximum(acc, 0.0) * mask_ref[...]


def _init_mm(f_bonds, W_i, br=4096):
    nb, k = f_bonds.shape
    return pl.pallas_call(
        _init_mm_kernel,
        grid=(nb // br,),
        in_specs=[
            pl.BlockSpec((br, k), lambda i: (i, 0)),
            pl.BlockSpec((k, H), lambda i: (0, 0)),
        ],
        out_specs=pl.BlockSpec((br, H), lambda i: (i, 0)),
        out_shape=jax.ShapeDtypeStruct((nb, H), jnp.float32),
    )(f_bonds, W_i)


def _reduce_mm(nei, br=4096):
    n = nei.shape[0]
    return pl.pallas_call(
        _reduce_kernel,
        grid=(n // br,),
        in_specs=[pl.BlockSpec((br, H), lambda i: (i, 0))],
        out_specs=pl.BlockSpec((br // MAX_NB, H), lambda i: (i, 0)),
        out_shape=jax.ShapeDtypeStruct((n // MAX_NB, H), jnp.float32),
    )(nei)


def _layer_mm(f_bonds, ga, gr, W_i, W_h16, br=4096):
    nb, k = f_bonds.shape
    return pl.pallas_call(
        _layer_mm_kernel,
        grid=(nb // br,),
        in_specs=[
            pl.BlockSpec((br, k), lambda i: (i, 0)),
            pl.BlockSpec((br, H), lambda i: (i, 0)),
            pl.BlockSpec((br, H), lambda i: (i, 0)),
            pl.BlockSpec((k, H), lambda i: (0, 0)),
            pl.BlockSpec((H, H), lambda i: (0, 0)),
        ],
        out_specs=pl.BlockSpec((br, H), lambda i: (i, 0)),
        out_shape=jax.ShapeDtypeStruct((nb, H), jnp.float32),
    )(f_bonds, ga, gr, W_i, W_h16)


def _final_mm(f_atoms, a_msg, W_o, mask, br=2000):
    na, fd = f_atoms.shape
    return pl.pallas_call(
        _final_mm_kernel,
        grid=(na // br,),
        in_specs=[
            pl.BlockSpec((br, fd), lambda i: (i, 0)),
            pl.BlockSpec((br, H), lambda i: (i, 0)),
            pl.BlockSpec((fd, H), lambda i: (0, 0)),
            pl.BlockSpec((H, H), lambda i: (0, 0)),
            pl.BlockSpec((br, 1), lambda i: (i, 0)),
        ],
        out_specs=pl.BlockSpec((br, H), lambda i: (i, 0)),
        out_shape=jax.ShapeDtypeStruct((na, H), jnp.float32),
    )(f_atoms, a_msg, W_o[:fd], W_o[fd:].astype(jnp.bfloat16), mask)


def _gs_body(msg_hbm, a2b_hbm, nei_hbm, idx_v, rows_v, *sems):
    wid = lax.axis_index("s") * NC + lax.axis_index("c")
    pltpu.sync_copy(a2b_hbm.at[wid], idx_v)

    def gidx(c):
        return idx_v.at[pl.ds(c * GS_CHUNK, GS_CHUNK)]

    sgs = sems[:GS_RING]
    sos = sems[GS_RING:]
    base = wid * (APW * MAX_NB)

    for b in range(GS_RING):
        pltpu.async_copy(msg_hbm.at[gidx(b)], rows_v.at[b], sgs[b])

    def chunk(c, b):
        pltpu.make_async_copy(
            msg_hbm.at[gidx(c)], rows_v.at[b], sgs[b]).wait()
        dst = pl.ds(base + c * GS_CHUNK, GS_CHUNK)
        pltpu.async_copy(rows_v.at[b], nei_hbm.at[dst], sos[b])
        pltpu.make_async_copy(rows_v.at[b], nei_hbm.at[dst], sos[b]).wait()
        pltpu.async_copy(msg_hbm.at[gidx(c + GS_RING)], rows_v.at[b], sgs[b])

    def step(s_, carry):
        for b in range(GS_RING):
            chunk(GS_RING * s_ + b, b)
        return carry

    nfull = GS_CHUNKS // GS_RING
    lax.fori_loop(0, nfull, step, 0)
    for q in range(GS_CHUNKS - nfull * GS_RING):
        chunk(nfull * GS_RING + q, q)
    for b in range(GS_RING):
        pltpu.make_async_copy(
            msg_hbm.at[gidx(b)], rows_v.at[b], sgs[b]).wait()


def _sc_gather_sum(message, a2b_r):
    nei = pl.kernel(
        _gs_body,
        out_type=jax.ShapeDtypeStruct((NAP * MAX_NB, H), jnp.float32),
        mesh=plsc.VectorSubcoreMesh(core_axis_name="c", subcore_axis_name="s"),
        scratch_types=[
            pltpu.VMEM(((GS_CHUNKS + GS_RING) * GS_CHUNK,), jnp.int32),
            pltpu.VMEM((GS_RING, GS_CHUNK, H), jnp.float32),
        ] + [pltpu.SemaphoreType.DMA] * (2 * GS_RING),
    )(message, a2b_r)
    return _reduce_mm(nei)


def _edge_body(amsg_hbm, msg_hbm, b2a_hbm, b2r_hbm, ga_hbm, gr_hbm,
               idxa_v, idxr_v, ga_v, gr_v, *sems):
    wid = lax.axis_index("s") * NC + lax.axis_index("c")
    pltpu.sync_copy(b2a_hbm.at[wid], idxa_v)
    pltpu.sync_copy(b2r_hbm.at[wid], idxr_v)

    def eidx(iv, c):
        return iv.at[pl.ds(c * ED_CHUNK, ED_CHUNK)]

    sas = sems[:ED_RING]
    srs = sems[ED_RING:2 * ED_RING]
    soa = sems[2 * ED_RING:3 * ED_RING]
    sor = sems[3 * ED_RING:]
    base = wid * BPW

    def g_in(c, b):
        pltpu.async_copy(amsg_hbm.at[eidx(idxa_v, c)], ga_v.at[b], sas[b])
        pltpu.async_copy(msg_hbm.at[eidx(idxr_v, c)], gr_v.at[b], srs[b])

    for b in range(ED_RING):
        g_in(b, b)

    def chunk(c, b):
        pltpu.make_async_copy(
            amsg_hbm.at[eidx(idxa_v, c)], ga_v.at[b], sas[b]).wait()
        pltpu.make_async_copy(
            msg_hbm.at[eidx(idxr_v, c)], gr_v.at[b], srs[b]).wait()
        dst = pl.ds(base + c * ED_CHUNK, ED_CHUNK)
        pltpu.async_copy(ga_v.at[b], ga_hbm.at[dst], soa[b])
        pltpu.async_copy(gr_v.at[b], gr_hbm.at[dst], sor[b])
        pltpu.make_async_copy(ga_v.at[b], ga_hbm.at[dst], soa[b]).wait()
        pltpu.make_async_copy(gr_v.at[b], gr_hbm.at[dst], sor[b]).wait()
        g_in(c + ED_RING, b)

    def step(s_, carry):
        for b in range(ED_RING):
            chunk(ED_RING * s_ + b, b)
        return carry

    nfull = ED_CHUNKS // ED_RING
    lax.fori_loop(0, nfull, step, 0)
    for q in range(ED_CHUNKS - nfull * ED_RING):
        chunk(nfull * ED_RING + q, q)
    for b in range(ED_RING):
        pltpu.make_async_copy(
            amsg_hbm.at[eidx(idxa_v, b)], ga_v.at[b], sas[b]).wait()
        pltpu.make_async_copy(
            msg_hbm.at[eidx(idxr_v, b)], gr_v.at[b], srs[b]).wait()


def _sc_edge(a_msg, message, b2a_r, b2r_r):
    return pl.kernel(
        _edge_body,
        out_type=[
            jax.ShapeDtypeStruct((NBP, H), jnp.float32),
            jax.ShapeDtypeStruct((NBP, H), jnp.float32),
        ],
        mesh=plsc.VectorSubcoreMesh(core_axis_name="c", subcore_axis_name="s"),
        scratch_types=[
            pltpu.VMEM(((ED_CHUNKS + ED_RING) * ED_CHUNK,), jnp.int32),
            pltpu.VMEM(((ED_CHUNKS + ED_RING) * ED_CHUNK,), jnp.int32),
            pltpu.VMEM((ED_RING, ED_CHUNK, H), jnp.float32),
            pltpu.VMEM((ED_RING, ED_CHUNK, H), jnp.float32),
        ] + [pltpu.SemaphoreType.DMA] * (4 * ED_RING),
    )(a_msg, message, b2a_r, b2r_r)


def kernel(f_atoms, f_bonds, a2b, b2a, b2revb, mask, W_i, W_h, W_o):
    a2b = a2b.astype(jnp.int32)
    b2a = b2a.astype(jnp.int32)
    b2revb = b2revb.astype(jnp.int32)

    def bond_remap(idx):
        return (idx // 10000) * BPW + (idx % 10000)

    f_bonds_p = jnp.pad(f_bonds.reshape(NW, 10000, -1),
                        ((0, 0), (0, BPW - 10000), (0, 0))).reshape(NBP, -1)
    a2b_v = bond_remap(a2b)
    b2r_v = bond_remap(b2revb)

    a2b_pad = jnp.zeros((NAP, MAX_NB), jnp.int32).at[:NA].set(a2b_v)
    a2b_r = jnp.pad(a2b_pad.reshape(NW, GS_CHUNKS * GS_CHUNK),
                    ((0, 0), (0, GS_RING * GS_CHUNK)))
    b2a_p = jnp.pad(b2a.reshape(NW, 10000), ((0, 0), (0, BPW - 10000)))
    b2a_r = jnp.pad(b2a_p, ((0, 0), (0, ED_RING * ED_CHUNK)))
    b2r_p = jnp.pad(b2r_v.reshape(NW, 10000), ((0, 0), (0, BPW - 10000)))
    b2r_r = jnp.pad(b2r_p, ((0, 0), (0, ED_RING * ED_CHUNK)))

    W_h16 = W_h.astype(jnp.bfloat16)
    message = _init_mm(f_bonds_p, W_i)
    for _ in range(DEPTH - 1):
        a_msg = _sc_gather_sum(message, a2b_r)
        ga, gr = _sc_edge(a_msg, message, b2a_r, b2r_r)
        message = _layer_mm(f_bonds_p, ga, gr, W_i, W_h16)
    a_msg = _sc_gather_sum(message, a2b_r)
    return _final_mm(f_atoms, a_msg[:NA], W_o, mask)


# speedup vs baseline: 1.3587x; 1.3587x over previous
import jax
import jax.numpy as jnp
from jax import lax
from jax.experimental import pallas as pl
from jax.experimental.pallas import tpu as pltpu
from jax.experimental.pallas import tpu_sc as plsc

H = 128
DEPTH = 4
NC, NS = 2, 16
NW = NC * NS

NB = 320000
NA = 10000
MAX_NB = 32
BPW = 10240
NBP = NW * BPW
APW = 320
NAP = NW * APW

GS_CHUNK = 256
GS_ATOMS = GS_CHUNK // MAX_NB
GS_CHUNKS = APW * MAX_NB // GS_CHUNK
GS_RING = 2
ED_CHUNK = 128
ED_CHUNKS = BPW // ED_CHUNK
ED_RING = 2


def _init_mm_kernel(fb_ref, wi_ref, out_ref):
    acc = jnp.dot(fb_ref[...], wi_ref[...], preferred_element_type=jnp.float32)
    out_ref[...] = jnp.maximum(acc, 0.0)


def _layer_mm_kernel(fb_ref, t_ref, wi_ref, wh_ref, out_ref):
    acc = jnp.dot(fb_ref[...], wi_ref[...], preferred_element_type=jnp.float32)
    acc = acc + jnp.dot(t_ref[...].astype(jnp.bfloat16), wh_ref[...],
                        preferred_element_type=jnp.float32)
    out_ref[...] = jnp.maximum(acc, 0.0)


def _final_mm_kernel(fa_ref, am_ref, wo1_ref, wo2_ref, mask_ref, out_ref):
    acc = jnp.dot(fa_ref[...], wo1_ref[...], preferred_element_type=jnp.float32)
    acc = acc + jnp.dot(am_ref[...].astype(jnp.bfloat16), wo2_ref[...],
                        preferred_element_type=jnp.float32)
    out_ref[...] = jnp.maximum(acc, 0.0) * mask_ref[...]


def _init_mm(f_bonds, W_i, br=4096):
    nb, k = f_bonds.shape
    return pl.pallas_call(
        _init_mm_kernel,
        grid=(nb // br,),
        in_specs=[
            pl.BlockSpec((br, k), lambda i: (i, 0)),
            pl.BlockSpec((k, H), lambda i: (0, 0)),
        ],
        out_specs=pl.BlockSpec((br, H), lambda i: (i, 0)),
        out_shape=jax.ShapeDtypeStruct((nb, H), jnp.float32),
    )(f_bonds, W_i)


def _layer_mm(f_bonds, t, W_i, W_h16, br=4096):
    nb, k = f_bonds.shape
    return pl.pallas_call(
        _layer_mm_kernel,
        grid=(nb // br,),
        in_specs=[
            pl.BlockSpec((br, k), lambda i: (i, 0)),
            pl.BlockSpec((br, H), lambda i: (i, 0)),
            pl.BlockSpec((k, H), lambda i: (0, 0)),
            pl.BlockSpec((H, H), lambda i: (0, 0)),
        ],
        out_specs=pl.BlockSpec((br, H), lambda i: (i, 0)),
        out_shape=jax.ShapeDtypeStruct((nb, H), jnp.float32),
    )(f_bonds, t, W_i, W_h16)


def _final_mm(f_atoms, a_msg, W_o, mask, br=2000):
    na, fd = f_atoms.shape
    return pl.pallas_call(
        _final_mm_kernel,
        grid=(na // br,),
        in_specs=[
            pl.BlockSpec((br, fd), lambda i: (i, 0)),
            pl.BlockSpec((br, H), lambda i: (i, 0)),
            pl.BlockSpec((fd, H), lambda i: (0, 0)),
            pl.BlockSpec((H, H), lambda i: (0, 0)),
            pl.BlockSpec((br, 1), lambda i: (i, 0)),
        ],
        out_specs=pl.BlockSpec((br, H), lambda i: (i, 0)),
        out_shape=jax.ShapeDtypeStruct((na, H), jnp.float32),
    )(f_atoms, a_msg, W_o[:fd], W_o[fd:].astype(jnp.bfloat16), mask)


def _gs_body(msg_hbm, a2b_hbm, amsg_hbm, idx_v, rows_v, out_v, *sems):
    wid = lax.axis_index("s") * NC + lax.axis_index("c")
    pltpu.sync_copy(a2b_hbm.at[wid], idx_v)

    def gidx(c):
        return idx_v.at[pl.ds(c * GS_CHUNK, GS_CHUNK)]

    for b in range(GS_RING):
        pltpu.async_copy(msg_hbm.at[gidx(b)], rows_v.at[b], sems[b])

    def step(s_, carry):
        for b in range(GS_RING):
            c = GS_RING * s_ + b
            pltpu.make_async_copy(
                msg_hbm.at[gidx(c)], rows_v.at[b], sems[b]).wait()

            def atom(i, carry2):
                for j in range(8):
                    acc = rows_v[b, i * 32, pl.ds(16 * j, 16)]
                    for k in range(1, 32):
                        acc = acc + rows_v[b, i * 32 + k, pl.ds(16 * j, 16)]
                    out_v[pl.ds((GS_ATOMS * c + i) * H + 16 * j, 16)] = acc
                return carry2

            lax.fori_loop(0, GS_ATOMS, atom, 0)
            pltpu.async_copy(
                msg_hbm.at[gidx(c + GS_RING)], rows_v.at[b], sems[b])
        return carry

    lax.fori_loop(0, GS_CHUNKS // GS_RING, step, 0)
    for b in range(GS_RING):
        pltpu.make_async_copy(
            msg_hbm.at[gidx(b)], rows_v.at[b], sems[b]).wait()
    pltpu.sync_copy(out_v, amsg_hbm.at[pl.ds(wid * APW * H, APW * H)])


def _sc_gather_sum(message, a2b_r):
    out_flat = pl.kernel(
        _gs_body,
        out_type=jax.ShapeDtypeStruct((NAP * H,), jnp.float32),
        mesh=plsc.VectorSubcoreMesh(core_axis_name="c", subcore_axis_name="s"),
        scratch_types=[
            pltpu.VMEM(((GS_CHUNKS + GS_RING) * GS_CHUNK,), jnp.int32),
            pltpu.VMEM((GS_RING, GS_CHUNK, H), jnp.float32),
            pltpu.VMEM((APW * H,), jnp.float32),
        ] + [pltpu.SemaphoreType.DMA] * GS_RING,
    )(message, a2b_r)
    return out_flat.reshape(NAP, H)


def _edge_body(amsg_hbm, msg_hbm, b2a_hbm, b2r_hbm, t_hbm, dummy_hbm,
               idxa_v, idxr_v, ga_v, gr_v, to_v, *sems):
    wid = lax.axis_index("s") * NC + lax.axis_index("c")
    pltpu.sync_copy(b2a_hbm.at[wid], idxa_v)
    pltpu.sync_copy(b2r_hbm.at[wid], idxr_v)

    def eidx(iv, c):
        return iv.at[pl.ds(c * ED_CHUNK, ED_CHUNK)]

    sas = sems[:ED_RING]
    srs = sems[ED_RING:2 * ED_RING]
    sos = sems[2 * ED_RING:]
    base = wid * BPW
    for b in range(ED_RING):
        pltpu.async_copy(amsg_hbm.at[eidx(idxa_v, b)], ga_v.at[b], sas[b])
        pltpu.async_copy(msg_hbm.at[eidx(idxr_v, b)], gr_v.at[b], srs[b])
        pltpu.async_copy(to_v.at[b], dummy_hbm.at[wid], sos[b])

    def step(s_, carry):
        for b in range(ED_RING):
            c = ED_RING * s_ + b
            pltpu.make_async_copy(
                amsg_hbm.at[eidx(idxa_v, c)], ga_v.at[b], sas[b]).wait()
            pltpu.make_async_copy(
                msg_hbm.at[eidx(idxr_v, c)], gr_v.at[b], srs[b]).wait()
            pltpu.make_async_copy(to_v.at[b], dummy_hbm.at[wid], sos[b]).wait()

            def row(r, carry2):
                for j in range(8):
                    to_v[b, r, pl.ds(16 * j, 16)] = (
                        ga_v[b, r, pl.ds(16 * j, 16)]
                        - gr_v[b, r, pl.ds(16 * j, 16)])
                return carry2

            lax.fori_loop(0, ED_CHUNK, row, 0)
            pltpu.async_copy(
                to_v.at[b],
                t_hbm.at[pl.ds(base + c * ED_CHUNK, ED_CHUNK)], sos[b])
            pltpu.async_copy(
                amsg_hbm.at[eidx(idxa_v, c + ED_RING)], ga_v.at[b], sas[b])
            pltpu.async_copy(
                msg_hbm.at[eidx(idxr_v, c + ED_RING)], gr_v.at[b], srs[b])
        return carry

    lax.fori_loop(0, ED_CHUNKS // ED_RING, step, 0)
    for b in range(ED_RING):
        pltpu.make_async_copy(
            amsg_hbm.at[eidx(idxa_v, b)], ga_v.at[b], sas[b]).wait()
        pltpu.make_async_copy(
            msg_hbm.at[eidx(idxr_v, b)], gr_v.at[b], srs[b]).wait()
        pltpu.make_async_copy(to_v.at[b], dummy_hbm.at[wid], sos[b]).wait()


def _sc_edge(a_msg, message, b2a_r, b2r_r):
    t, _ = pl.kernel(
        _edge_body,
        out_type=[
            jax.ShapeDtypeStruct((NBP, H), jnp.float32),
            jax.ShapeDtypeStruct((NW, ED_CHUNK, H), jnp.float32),
        ],
        mesh=plsc.VectorSubcoreMesh(core_axis_name="c", subcore_axis_name="s"),
        scratch_types=[
            pltpu.VMEM(((ED_CHUNKS + ED_RING) * ED_CHUNK,), jnp.int32),
            pltpu.VMEM(((ED_CHUNKS + ED_RING) * ED_CHUNK,), jnp.int32),
            pltpu.VMEM((ED_RING, ED_CHUNK, H), jnp.float32),
            pltpu.VMEM((ED_RING, ED_CHUNK, H), jnp.float32),
            pltpu.VMEM((ED_RING, ED_CHUNK, H), jnp.float32),
        ] + [pltpu.SemaphoreType.DMA] * (3 * ED_RING),
    )(a_msg, message, b2a_r, b2r_r)
    return t


def kernel(f_atoms, f_bonds, a2b, b2a, b2revb, mask, W_i, W_h, W_o):
    a2b = a2b.astype(jnp.int32)
    b2a = b2a.astype(jnp.int32)
    b2revb = b2revb.astype(jnp.int32)

    def bond_remap(idx):
        return (idx // 10000) * BPW + (idx % 10000)

    f_bonds_p = jnp.pad(f_bonds.reshape(NW, 10000, -1),
                        ((0, 0), (0, BPW - 10000), (0, 0))).reshape(NBP, -1)
    a2b_v = bond_remap(a2b)
    b2r_v = bond_remap(b2revb)

    a2b_pad = jnp.zeros((NAP, MAX_NB), jnp.int32).at[:NA].set(a2b_v)
    a2b_r = jnp.pad(a2b_pad.reshape(NW, GS_CHUNKS * GS_CHUNK),
                    ((0, 0), (0, GS_RING * GS_CHUNK)))
    b2a_p = jnp.pad(b2a.reshape(NW, 10000), ((0, 0), (0, BPW - 10000)))
    b2a_r = jnp.pad(b2a_p, ((0, 0), (0, ED_RING * ED_CHUNK)))
    b2r_p = jnp.pad(b2r_v.reshape(NW, 10000), ((0, 0), (0, BPW - 10000)))
    b2r_r = jnp.pad(b2r_p, ((0, 0), (0, ED_RING * ED_CHUNK)))

    W_h16 = W_h.astype(jnp.bfloat16)
    message = _init_mm(f_bonds_p, W_i)
    for _ in range(DEPTH - 1):
        a_msg = _sc_gather_sum(message, a2b_r)
        t = _sc_edge(a_msg, message, b2a_r, b2r_r)
        message = _layer_mm(f_bonds_p, t, W_i, W_h16)
    a_msg = _sc_gather_sum(message, a2b_r)
    return _final_mm(f_atoms, a_msg[:NA], W_o, mask)


# speedup vs baseline: 1.7973x; 1.3228x over previous
import jax
import jax.numpy as jnp
from jax import lax
from jax.experimental import pallas as pl
from jax.experimental.pallas import tpu as pltpu
from jax.experimental.pallas import tpu_sc as plsc

H = 128
DEPTH = 4
NC, NS = 2, 16
NW = NC * NS

NB = 320000
NA = 10000
NAP = NW * 320
MAX_NB = 32

GS_CHUNK = 64
GS_CHUNKS = 160
GS_RING = 4
ED_CHUNK = 40
ED_CHUNKS = 250
ED_RING = 5


def _init_mm_kernel(fb_ref, wi_ref, out_ref):
    acc = jnp.dot(fb_ref[...], wi_ref[...], preferred_element_type=jnp.float32)
    out_ref[...] = jnp.maximum(acc, 0.0)


def _layer_mm_kernel(fb_ref, t_ref, wi_ref, wh_ref, out_ref):
    acc = jnp.dot(fb_ref[...], wi_ref[...], preferred_element_type=jnp.float32)
    acc = acc + jnp.dot(t_ref[...].astype(jnp.bfloat16), wh_ref[...],
                        preferred_element_type=jnp.float32)
    out_ref[...] = jnp.maximum(acc, 0.0)


def _final_mm_kernel(fa_ref, am_ref, wo1_ref, wo2_ref, mask_ref, out_ref):
    acc = jnp.dot(fa_ref[...], wo1_ref[...], preferred_element_type=jnp.float32)
    acc = acc + jnp.dot(am_ref[...].astype(jnp.bfloat16), wo2_ref[...],
                        preferred_element_type=jnp.float32)
    out_ref[...] = jnp.maximum(acc, 0.0) * mask_ref[...]


def _init_mm(f_bonds, W_i, br=3200):
    nb, k = f_bonds.shape
    return pl.pallas_call(
        _init_mm_kernel,
        grid=(nb // br,),
        in_specs=[
            pl.BlockSpec((br, k), lambda i: (i, 0)),
            pl.BlockSpec((k, H), lambda i: (0, 0)),
        ],
        out_specs=pl.BlockSpec((br, H), lambda i: (i, 0)),
        out_shape=jax.ShapeDtypeStruct((nb, H), jnp.float32),
    )(f_bonds, W_i)


def _layer_mm(f_bonds, t, W_i, W_h16, br=3200):
    nb, k = f_bonds.shape
    return pl.pallas_call(
        _layer_mm_kernel,
        grid=(nb // br,),
        in_specs=[
            pl.BlockSpec((br, k), lambda i: (i, 0)),
            pl.BlockSpec((br, H), lambda i: (i, 0)),
            pl.BlockSpec((k, H), lambda i: (0, 0)),
            pl.BlockSpec((H, H), lambda i: (0, 0)),
        ],
        out_specs=pl.BlockSpec((br, H), lambda i: (i, 0)),
        out_shape=jax.ShapeDtypeStruct((nb, H), jnp.float32),
    )(f_bonds, t, W_i, W_h16)


def _final_mm(f_atoms, a_msg, W_o, mask, br=2000):
    na, fd = f_atoms.shape
    return pl.pallas_call(
        _final_mm_kernel,
        grid=(na // br,),
        in_specs=[
            pl.BlockSpec((br, fd), lambda i: (i, 0)),
            pl.BlockSpec((br, H), lambda i: (i, 0)),
            pl.BlockSpec((fd, H), lambda i: (0, 0)),
            pl.BlockSpec((H, H), lambda i: (0, 0)),
            pl.BlockSpec((br, 1), lambda i: (i, 0)),
        ],
        out_specs=pl.BlockSpec((br, H), lambda i: (i, 0)),
        out_shape=jax.ShapeDtypeStruct((na, H), jnp.float32),
    )(f_atoms, a_msg, W_o[:fd], W_o[fd:].astype(jnp.bfloat16), mask)


def _gs_body(msg_hbm, a2b_hbm, amsg_hbm, idx_v, rows_v, out_v, *sems):
    wid = lax.axis_index("s") * NC + lax.axis_index("c")
    pltpu.sync_copy(a2b_hbm.at[wid], idx_v)

    def gidx(c):
        return idx_v.at[pl.ds(c * GS_CHUNK, GS_CHUNK)]

    for b in range(GS_RING):
        pltpu.async_copy(msg_hbm.at[gidx(b)], rows_v.at[b], sems[b])

    def step(s, carry):
        for b in range(GS_RING):
            c = GS_RING * s + b
            pltpu.make_async_copy(
                msg_hbm.at[gidx(c)], rows_v.at[b], sems[b]).wait()
            for i in range(2):
                for j in range(8):
                    acc = rows_v[b, 32 * i, pl.ds(16 * j, 16)]
                    for k in range(1, 32):
                        acc = acc + rows_v[b, 32 * i + k, pl.ds(16 * j, 16)]
                    out_v[pl.ds((2 * c + i) * H + 16 * j, 16)] = acc
            pltpu.async_copy(
                msg_hbm.at[gidx(c + GS_RING)], rows_v.at[b], sems[b])
        return carry

    lax.fori_loop(0, GS_CHUNKS // GS_RING, step, 0)
    for b in range(GS_RING):
        pltpu.make_async_copy(
            msg_hbm.at[gidx(b)], rows_v.at[b], sems[b]).wait()
    pltpu.sync_copy(out_v, amsg_hbm.at[pl.ds(wid * 320 * H, 320 * H)])


def _sc_gather_sum(message, a2b_r):
    out_flat = pl.kernel(
        _gs_body,
        out_type=jax.ShapeDtypeStruct((NAP * H,), jnp.float32),
        mesh=plsc.VectorSubcoreMesh(core_axis_name="c", subcore_axis_name="s"),
        scratch_types=[
            pltpu.VMEM(((GS_CHUNKS + GS_RING) * GS_CHUNK,), jnp.int32),
            pltpu.VMEM((GS_RING, GS_CHUNK, H), jnp.float32),
            pltpu.VMEM((320 * H,), jnp.float32),
        ] + [pltpu.SemaphoreType.DMA] * GS_RING,
    )(message, a2b_r)
    return out_flat.reshape(NAP, H)


def _edge_body(amsg_hbm, msg_hbm, b2a_hbm, b2r_hbm, t_hbm, dummy_hbm,
               idxa_v, idxr_v, ga_v, gr_v, to_v, *sems):
    wid = lax.axis_index("s") * NC + lax.axis_index("c")
    pltpu.sync_copy(b2a_hbm.at[wid], idxa_v)
    pltpu.sync_copy(b2r_hbm.at[wid], idxr_v)

    def eidx(iv, c):
        return iv.at[pl.ds(c * ED_CHUNK, ED_CHUNK)]

    sas = sems[:ED_RING]
    srs = sems[ED_RING:2 * ED_RING]
    sos = sems[2 * ED_RING:]
    base = wid * 10000
    for b in range(ED_RING):
        pltpu.async_copy(amsg_hbm.at[eidx(idxa_v, b)], ga_v.at[b], sas[b])
        pltpu.async_copy(msg_hbm.at[eidx(idxr_v, b)], gr_v.at[b], srs[b])
        pltpu.async_copy(to_v.at[b], dummy_hbm.at[wid], sos[b])

    def step(s, carry):
        for b in range(ED_RING):
            c = ED_RING * s + b
            pltpu.make_async_copy(
                amsg_hbm.at[eidx(idxa_v, c)], ga_v.at[b], sas[b]).wait()
            pltpu.make_async_copy(
                msg_hbm.at[eidx(idxr_v, c)], gr_v.at[b], srs[b]).wait()
            pltpu.make_async_copy(to_v.at[b], dummy_hbm.at[wid], sos[b]).wait()
            for r in range(ED_CHUNK):
                for j in range(8):
                    to_v[b, r, pl.ds(16 * j, 16)] = (
                        ga_v[b, r, pl.ds(16 * j, 16)]
                        - gr_v[b, r, pl.ds(16 * j, 16)])
            pltpu.async_copy(
                to_v.at[b],
                t_hbm.at[pl.ds(base + c * ED_CHUNK, ED_CHUNK)], sos[b])
            pltpu.async_copy(
                amsg_hbm.at[eidx(idxa_v, c + ED_RING)], ga_v.at[b], sas[b])
            pltpu.async_copy(
                msg_hbm.at[eidx(idxr_v, c + ED_RING)], gr_v.at[b], srs[b])
        return carry

    lax.fori_loop(0, ED_CHUNKS // ED_RING, step, 0)
    for b in range(ED_RING):
        pltpu.make_async_copy(
            amsg_hbm.at[eidx(idxa_v, b)], ga_v.at[b], sas[b]).wait()
        pltpu.make_async_copy(
            msg_hbm.at[eidx(idxr_v, b)], gr_v.at[b], srs[b]).wait()
        pltpu.make_async_copy(to_v.at[b], dummy_hbm.at[wid], sos[b]).wait()


def _sc_edge(a_msg, message, b2a_r, b2r_r):
    t, _ = pl.kernel(
        _edge_body,
        out_type=[
            jax.ShapeDtypeStruct((NB, H), jnp.float32),
            jax.ShapeDtypeStruct((NW, ED_CHUNK, H), jnp.float32),
        ],
        mesh=plsc.VectorSubcoreMesh(core_axis_name="c", subcore_axis_name="s"),
        scratch_types=[
            pltpu.VMEM(((ED_CHUNKS + ED_RING) * ED_CHUNK,), jnp.int32),
            pltpu.VMEM(((ED_CHUNKS + ED_RING) * ED_CHUNK,), jnp.int32),
            pltpu.VMEM((ED_RING, ED_CHUNK, H), jnp.float32),
            pltpu.VMEM((ED_RING, ED_CHUNK, H), jnp.float32),
            pltpu.VMEM((ED_RING, ED_CHUNK, H), jnp.float32),
        ] + [pltpu.SemaphoreType.DMA] * (3 * ED_RING),
    )(a_msg, message, b2a_r, b2r_r)
    return t


def kernel(f_atoms, f_bonds, a2b, b2a, b2revb, mask, W_i, W_h, W_o):
    a2b = a2b.astype(jnp.int32)
    b2a = b2a.astype(jnp.int32)
    b2revb = b2revb.astype(jnp.int32)

    a2b_pad = jnp.zeros((NAP, MAX_NB), jnp.int32).at[:NA].set(a2b)
    a2b_r = jnp.pad(a2b_pad.reshape(NW, GS_CHUNKS * GS_CHUNK),
                    ((0, 0), (0, GS_RING * GS_CHUNK)))
    b2a_r = jnp.pad(b2a.reshape(NW, ED_CHUNKS * ED_CHUNK),
                    ((0, 0), (0, ED_RING * ED_CHUNK)))
    b2r_r = jnp.pad(b2revb.reshape(NW, ED_CHUNKS * ED_CHUNK),
                    ((0, 0), (0, ED_RING * ED_CHUNK)))

    W_h16 = W_h.astype(jnp.bfloat16)
    message = _init_mm(f_bonds, W_i)
    for _ in range(DEPTH - 1):
        a_msg = _sc_gather_sum(message, a2b_r)
        t = _sc_edge(a_msg, message, b2a_r, b2r_r)
        message = _layer_mm(f_bonds, t, W_i, W_h16)
    a_msg = _sc_gather_sum(message, a2b_r)
    return _final_mm(f_atoms, a_msg[:NA], W_o, mask)


# speedup vs baseline: 1.7973x; 1.0000x over previous
import jax
import jax.numpy as jnp
from jax import lax
from jax.experimental import pallas as pl
from jax.experimental.pallas import tpu as pltpu
from jax.experimental.pallas import tpu_sc as plsc

H = 128
DEPTH = 4
NC, NS = 2, 16
NW = NC * NS

NB = 320000
NA = 10000
NAP = NW * 320
MAX_NB = 32

GS_CHUNK = 128
GS_CHUNKS = 80
GS_RING = 2
ED_CHUNK = 40
ED_CHUNKS = 250
ED_RING = 5


def _init_mm_kernel(fb_ref, wi_ref, out_ref):
    acc = jnp.dot(fb_ref[...], wi_ref[...], preferred_element_type=jnp.float32)
    out_ref[...] = jnp.maximum(acc, 0.0)


def _layer_mm_kernel(fb_ref, t_ref, wi_ref, wh_ref, out_ref):
    acc = jnp.dot(fb_ref[...], wi_ref[...], preferred_element_type=jnp.float32)
    acc = acc + jnp.dot(t_ref[...].astype(jnp.bfloat16), wh_ref[...],
                        preferred_element_type=jnp.float32)
    out_ref[...] = jnp.maximum(acc, 0.0)


def _final_mm_kernel(fa_ref, am_ref, wo1_ref, wo2_ref, mask_ref, out_ref):
    acc = jnp.dot(fa_ref[...], wo1_ref[...], preferred_element_type=jnp.float32)
    acc = acc + jnp.dot(am_ref[...].astype(jnp.bfloat16), wo2_ref[...],
                        preferred_element_type=jnp.float32)
    out_ref[...] = jnp.maximum(acc, 0.0) * mask_ref[...]


def _init_mm(f_bonds, W_i, br=3200):
    nb, k = f_bonds.shape
    return pl.pallas_call(
        _init_mm_kernel,
        grid=(nb // br,),
        in_specs=[
            pl.BlockSpec((br, k), lambda i: (i, 0)),
            pl.BlockSpec((k, H), lambda i: (0, 0)),
        ],
        out_specs=pl.BlockSpec((br, H), lambda i: (i, 0)),
        out_shape=jax.ShapeDtypeStruct((nb, H), jnp.float32),
    )(f_bonds, W_i)


def _layer_mm(f_bonds, t, W_i, W_h16, br=3200):
    nb, k = f_bonds.shape
    return pl.pallas_call(
        _layer_mm_kernel,
        grid=(nb // br,),
        in_specs=[
            pl.BlockSpec((br, k), lambda i: (i, 0)),
            pl.BlockSpec((br, H), lambda i: (i, 0)),
            pl.BlockSpec((k, H), lambda i: (0, 0)),
            pl.BlockSpec((H, H), lambda i: (0, 0)),
        ],
        out_specs=pl.BlockSpec((br, H), lambda i: (i, 0)),
        out_shape=jax.ShapeDtypeStruct((nb, H), jnp.float32),
    )(f_bonds, t, W_i, W_h16)


def _final_mm(f_atoms, a_msg, W_o, mask, br=2000):
    na, fd = f_atoms.shape
    return pl.pallas_call(
        _final_mm_kernel,
        grid=(na // br,),
        in_specs=[
            pl.BlockSpec((br, fd), lambda i: (i, 0)),
            pl.BlockSpec((br, H), lambda i: (i, 0)),
            pl.BlockSpec((fd, H), lambda i: (0, 0)),
            pl.BlockSpec((H, H), lambda i: (0, 0)),
            pl.BlockSpec((br, 1), lambda i: (i, 0)),
        ],
        out_specs=pl.BlockSpec((br, H), lambda i: (i, 0)),
        out_shape=jax.ShapeDtypeStruct((na, H), jnp.float32),
    )(f_atoms, a_msg, W_o[:fd], W_o[fd:].astype(jnp.bfloat16), mask)


def _gs_body(msg_hbm, a2b_hbm, amsg_hbm, idx_v, rows_v, out_v, *sems):
    wid = lax.axis_index("s") * NC + lax.axis_index("c")
    pltpu.sync_copy(a2b_hbm.at[wid], idx_v)

    def gidx(c):
        return idx_v.at[pl.ds(c * GS_CHUNK, GS_CHUNK)]

    for b in range(GS_RING):
        pltpu.async_copy(msg_hbm.at[gidx(b)], rows_v.at[b], sems[b])

    def step(s, carry):
        for b in range(GS_RING):
            c = GS_RING * s + b
            pltpu.make_async_copy(
                msg_hbm.at[gidx(c)], rows_v.at[b], sems[b]).wait()
            for i in range(4):
                for j in range(8):
                    acc = rows_v[b, 32 * i, pl.ds(16 * j, 16)]
                    for k in range(1, 32):
                        acc = acc + rows_v[b, 32 * i + k, pl.ds(16 * j, 16)]
                    out_v[pl.ds((4 * c + i) * H + 16 * j, 16)] = acc
            pltpu.async_copy(
                msg_hbm.at[gidx(c + GS_RING)], rows_v.at[b], sems[b])
        return carry

    lax.fori_loop(0, GS_CHUNKS // GS_RING, step, 0)
    for b in range(GS_RING):
        pltpu.make_async_copy(
            msg_hbm.at[gidx(b)], rows_v.at[b], sems[b]).wait()
    pltpu.sync_copy(out_v, amsg_hbm.at[pl.ds(wid * 320 * H, 320 * H)])


def _sc_gather_sum(message, a2b_r):
    out_flat = pl.kernel(
        _gs_body,
        out_type=jax.ShapeDtypeStruct((NAP * H,), jnp.float32),
        mesh=plsc.VectorSubcoreMesh(core_axis_name="c", subcore_axis_name="s"),
        scratch_types=[
            pltpu.VMEM(((GS_CHUNKS + GS_RING) * GS_CHUNK,), jnp.int32),
            pltpu.VMEM((GS_RING, GS_CHUNK, H), jnp.float32),
            pltpu.VMEM((320 * H,), jnp.float32),
        ] + [pltpu.SemaphoreType.DMA] * GS_RING,
    )(message, a2b_r)
    return out_flat.reshape(NAP, H)


def _edge_body(amsg_hbm, msg_hbm, b2a_hbm, b2r_hbm, t_hbm, dummy_hbm,
               idxa_v, idxr_v, ga_v, gr_v, to_v, *sems):
    wid = lax.axis_index("s") * NC + lax.axis_index("c")
    pltpu.sync_copy(b2a_hbm.at[wid], idxa_v)
    pltpu.sync_copy(b2r_hbm.at[wid], idxr_v)

    def eidx(iv, c):
        return iv.at[pl.ds(c * ED_CHUNK, ED_CHUNK)]

    sas = sems[:ED_RING]
    srs = sems[ED_RING:2 * ED_RING]
    sos = sems[2 * ED_RING:]
    base = wid * 10000
    for b in range(ED_RING):
        pltpu.async_copy(amsg_hbm.at[eidx(idxa_v, b)], ga_v.at[b], sas[b])
        pltpu.async_copy(msg_hbm.at[eidx(idxr_v, b)], gr_v.at[b], srs[b])
        pltpu.async_copy(to_v.at[b], dummy_hbm.at[wid], sos[b])

    def step(s, carry):
        for b in range(ED_RING):
            c = ED_RING * s + b
            pltpu.make_async_copy(
                amsg_hbm.at[eidx(idxa_v, c)], ga_v.at[b], sas[b]).wait()
            pltpu.make_async_copy(
                msg_hbm.at[eidx(idxr_v, c)], gr_v.at[b], srs[b]).wait()
            pltpu.make_async_copy(to_v.at[b], dummy_hbm.at[wid], sos[b]).wait()
            for r in range(ED_CHUNK):
                for j in range(8):
                    to_v[b, r, pl.ds(16 * j, 16)] = (
                        ga_v[b, r, pl.ds(16 * j, 16)]
                        - gr_v[b, r, pl.ds(16 * j, 16)])
            pltpu.async_copy(
                to_v.at[b],
                t_hbm.at[pl.ds(base + c * ED_CHUNK, ED_CHUNK)], sos[b])
            pltpu.async_copy(
                amsg_hbm.at[eidx(idxa_v, c + ED_RING)], ga_v.at[b], sas[b])
            pltpu.async_copy(
                msg_hbm.at[eidx(idxr_v, c + ED_RING)], gr_v.at[b], srs[b])
        return carry

    lax.fori_loop(0, ED_CHUNKS // ED_RING, step, 0)
    for b in range(ED_RING):
        pltpu.make_async_copy(
            amsg_hbm.at[eidx(idxa_v, b)], ga_v.at[b], sas[b]).wait()
        pltpu.make_async_copy(
            msg_hbm.at[eidx(idxr_v, b)], gr_v.at[b], srs[b]).wait()
        pltpu.make_async_copy(to_v.at[b], dummy_hbm.at[wid], sos[b]).wait()


def _sc_edge(a_msg, message, b2a_r, b2r_r):
    t, _ = pl.kernel(
        _edge_body,
        out_type=[
            jax.ShapeDtypeStruct((NB, H), jnp.float32),
            jax.ShapeDtypeStruct((NW, ED_CHUNK, H), jnp.float32),
        ],
        mesh=plsc.VectorSubcoreMesh(core_axis_name="c", subcore_axis_name="s"),
        scratch_types=[
            pltpu.VMEM(((ED_CHUNKS + ED_RING) * ED_CHUNK,), jnp.int32),
            pltpu.VMEM(((ED_CHUNKS + ED_RING) * ED_CHUNK,), jnp.int32),
            pltpu.VMEM((ED_RING, ED_CHUNK, H), jnp.float32),
            pltpu.VMEM((ED_RING, ED_CHUNK, H), jnp.float32),
            pltpu.VMEM((ED_RING, ED_CHUNK, H), jnp.float32),
        ] + [pltpu.SemaphoreType.DMA] * (3 * ED_RING),
    )(a_msg, message, b2a_r, b2r_r)
    return t


def kernel(f_atoms, f_bonds, a2b, b2a, b2revb, mask, W_i, W_h, W_o):
    a2b = a2b.astype(jnp.int32)
    b2a = b2a.astype(jnp.int32)
    b2revb = b2revb.astype(jnp.int32)

    a2b_pad = jnp.zeros((NAP, MAX_NB), jnp.int32).at[:NA].set(a2b)
    a2b_r = jnp.pad(a2b_pad.reshape(NW, GS_CHUNKS * GS_CHUNK),
                    ((0, 0), (0, GS_RING * GS_CHUNK)))
    b2a_r = jnp.pad(b2a.reshape(NW, ED_CHUNKS * ED_CHUNK),
                    ((0, 0), (0, ED_RING * ED_CHUNK)))
    b2r_r = jnp.pad(b2revb.reshape(NW, ED_CHUNKS * ED_CHUNK),
                    ((0, 0), (0, ED_RING * ED_CHUNK)))

    W_h16 = W_h.astype(jnp.bfloat16)
    message = _init_mm(f_bonds, W_i)
    for _ in range(DEPTH - 1):
        a_msg = _sc_gather_sum(message, a2b_r)
        t = _sc_edge(a_msg, message, b2a_r, b2r_r)
        message = _layer_mm(f_bonds, t, W_i, W_h16)
    a_msg = _sc_gather_sum(message, a2b_r)
    return _final_mm(f_atoms, a_msg[:NA], W_o, mask)
